# baseline (speedup 1.0000x reference)
import jax
import jax.numpy as jnp
from jax import lax
from jax.experimental import pallas as pl
from jax.experimental.pallas import tpu as pltpu
from jax.experimental.pallas import tpu_sc as plsc

_B, _N, _FD = 8, 4096, 128
_M, _K = 1024, 32
_GH, _FH = 256, 256

_ROWS = 512


def _fps_body(x_ref, y_ref, z_ref, cx_ref, cy_ref, cz_ref):
    X = x_ref[...]
    Y = y_ref[...]
    Z = z_ref[...]
    lane = jax.lax.broadcasted_iota(jnp.int32, (_B, _N), 1)
    big = jnp.int32(_N)

    mlane = jax.lax.broadcasted_iota(jnp.int32, (_B, _M), 1)

    first = lane == 0
    cx0 = jnp.sum(jnp.where(first, X, 0.0), axis=1, keepdims=True)
    cy0 = jnp.sum(jnp.where(first, Y, 0.0), axis=1, keepdims=True)
    cz0 = jnp.sum(jnp.where(first, Z, 0.0), axis=1, keepdims=True)
    zerosm = jnp.zeros((_B, _M), jnp.float32)
    cxs0 = jnp.where(mlane == 0, cx0, zerosm)
    cys0 = jnp.where(mlane == 0, cy0, zerosm)
    czs0 = jnp.where(mlane == 0, cz0, zerosm)

    dists0 = jnp.full((_B, _N), 1e10, dtype=jnp.float32)

    def body(i, state):
        dists, lx, ly, lz, cxs, cys, czs = state
        d = (X - lx) ** 2 + (Y - ly) ** 2 + (Z - lz) ** 2
        dists = jnp.minimum(dists, d)
        m = jnp.max(dists, axis=1, keepdims=True)
        jstar = jnp.min(jnp.where(dists == m, lane, big), axis=1, keepdims=True)
        sel = lane == jstar
        nx = jnp.sum(jnp.where(sel, X, 0.0), axis=1, keepdims=True)
        ny = jnp.sum(jnp.where(sel, Y, 0.0), axis=1, keepdims=True)
        nz = jnp.sum(jnp.where(sel, Z, 0.0), axis=1, keepdims=True)
        here = mlane == i
        cxs = jnp.where(here, nx, cxs)
        cys = jnp.where(here, ny, cys)
        czs = jnp.where(here, nz, czs)
        return dists, nx, ny, nz, cxs, cys, czs

    _, _, _, _, cxs, cys, czs = jax.lax.fori_loop(
        1, _M, body, (dists0, cx0, cy0, cz0, cxs0, cys0, czs0))
    cx_ref[...] = cxs
    cy_ref[...] = cys
    cz_ref[...] = czs


def _fps_centroids(xyz):
    x = xyz[:, :, 0]
    y = xyz[:, :, 1]
    z = xyz[:, :, 2]
    shp = jax.ShapeDtypeStruct((_B, _M), jnp.float32)
    cx, cy, cz = pl.pallas_call(
        _fps_body,
        out_shape=(shp, shp, shp),
    )(x, y, z)
    return jnp.stack([cx, cy, cz], axis=-1)


_NW = 32
_IPW = (_B * _M * _K) // _NW
_FCH = 256
_CCH = 2048


def _gather_body(feat_hbm, xp_hbm, yp_hbm, zp_hbm, idx_hbm,
                 xf_out, gx_out, gy_out, gz_out,
                 idx2_v, gx_v, gy_v, gz_v, rows_v, sem, semf):
    wid = lax.axis_index("s") * 2 + lax.axis_index("c")
    rows_per_w = _IPW // 128
    r0 = wid * rows_per_w

    def super_chunk(c, carry):
        rbase = r0 + c * 16
        pltpu.sync_copy(idx_hbm.at[pl.ds(rbase, 16)], idx2_v)

        def j_body(j, carry2):
            iv = idx2_v.at[j]
            cg1 = pltpu.async_copy(xp_hbm.at[iv], gx_v.at[j], sem)
            cg2 = pltpu.async_copy(yp_hbm.at[iv], gy_v.at[j], sem)
            cg3 = pltpu.async_copy(zp_hbm.at[iv], gz_v.at[j], sem)
            cf = pltpu.async_copy(feat_hbm.at[iv], rows_v, semf)
            cg1.wait()
            cg2.wait()
            cg3.wait()
            cf.wait()
            pltpu.sync_copy(rows_v, xf_out.at[pl.ds((rbase + j) * 128, 128)])
            return carry2

        lax.fori_loop(0, 16, j_body, 0)
        pltpu.sync_copy(gx_v, gx_out.at[pl.ds(rbase, 16)])
        pltpu.sync_copy(gy_v, gy_out.at[pl.ds(rbase, 16)])
        pltpu.sync_copy(gz_v, gz_out.at[pl.ds(rbase, 16)])
        return carry

    lax.fori_loop(0, rows_per_w // 16, super_chunk, 0)


def _sc_gather(features_flat, xyz, idx2):
    mesh = plsc.VectorSubcoreMesh(core_axis_name="c", subcore_axis_name="s")
    nrows = _B * _M * _K
    k = pl.kernel(
        _gather_body,
        mesh=mesh,
        out_type=(
            jax.ShapeDtypeStruct((nrows, _FD), jnp.float32),
            jax.ShapeDtypeStruct((nrows // 128, 128), jnp.float32),
            jax.ShapeDtypeStruct((nrows // 128, 128), jnp.float32),
            jax.ShapeDtypeStruct((nrows // 128, 128), jnp.float32),
        ),
        scratch_types=[
            pltpu.VMEM((16, 128), jnp.int32),
            pltpu.VMEM((16, 128), jnp.float32),
            pltpu.VMEM((16, 128), jnp.float32),
            pltpu.VMEM((16, 128), jnp.float32),
            pltpu.VMEM((128, _FD), jnp.float32),
            pltpu.SemaphoreType.DMA,
            pltpu.SemaphoreType.DMA,
        ],
    )
    xflat = xyz[:, :, 0].reshape(_B * _N)
    yflat = xyz[:, :, 1].reshape(_B * _N)
    zflat = xyz[:, :, 2].reshape(_B * _N)
    return k(features_flat, xflat, yflat, zflat, idx2)


def _mlp_block(dx_ref, dy_ref, dz_ref, xf_ref, w1x_ref, w1y_ref, w1z_ref,
               w1e2_ref, b1g_ref, w2gt_ref, w2ge_ref, w1ft_ref, b1f_ref,
               w2ft_ref, b2f_ref, out_ref):
    dx = dx_ref[...]
    dy = dy_ref[...]
    dz = dz_ref[...]
    e2 = -0.5 * (dx * dx + dy * dy + dz * dz)
    h = (jnp.dot(dx, w1x_ref[...], preferred_element_type=jnp.float32)
         + jnp.dot(dy, w1y_ref[...], preferred_element_type=jnp.float32)
         + jnp.dot(dz, w1z_ref[...], preferred_element_type=jnp.float32)
         + jnp.dot(e2, w1e2_ref[...], preferred_element_type=jnp.float32)
         + b1g_ref[...])
    e2h = -0.5 * jnp.sum(h * h, axis=1, keepdims=True)
    xgeo = (jnp.dot(h, w2gt_ref[...], preferred_element_type=jnp.float32)
            - w2ge_ref[0, :][None, :] + e2h * w2ge_ref[1, :][None, :])
    hf = jnp.maximum(
        jnp.dot(xf_ref[...], w1ft_ref[...], preferred_element_type=jnp.float32)
        + b1f_ref[...], 0.0)
    xfeat = jnp.dot(hf, w2ft_ref[...], preferred_element_type=jnp.float32) + b2f_ref[...]
    out_ref[:, :_GH] = xgeo
    out_ref[:, _GH:] = xfeat


def _mlps(dx, dy, dz, xf, W1g, W2g, W1f, b1f, W2f, b2f):
    nrows = dx.shape[0]
    grid = nrows // _ROWS
    w1gt = W1g.T
    w1x = w1gt[0::5]
    w1y = w1gt[1::5]
    w1z = w1gt[2::5]
    w1e2 = w1gt[4::5]
    b1g = -jnp.sum(w1gt[3::5], axis=0, keepdims=True)
    w2gt = W2g[:, :_GH].T
    w2ge = W2g[:, _GH:].T
    kspec = pl.BlockSpec((_K, _GH), lambda i: (0, 0))
    bspec = pl.BlockSpec((1, _FH), lambda i: (0, 0))
    rkspec = pl.BlockSpec((_ROWS, _K), lambda i: (i, 0))
    out = pl.pallas_call(
        _mlp_block,
        grid=(grid,),
        in_specs=[
            rkspec, rkspec, rkspec,
            pl.BlockSpec((_ROWS, _K * _FD), lambda i: (i, 0)),
            kspec, kspec, kspec, kspec,
            bspec,
            pl.BlockSpec((_GH, _GH), lambda i: (0, 0)),
            pl.BlockSpec((2, _GH), lambda i: (0, 0)),
            pl.BlockSpec((_K * _FD, _FH), lambda i: (0, 0)),
            bspec,
            pl.BlockSpec((_FH, _FH), lambda i: (0, 0)),
            bspec,
        ],
        out_specs=pl.BlockSpec((_ROWS, _GH + _FH), lambda i: (i, 0)),
        out_shape=jax.ShapeDtypeStruct((nrows, _GH + _FH), jnp.float32),
    )(dx, dy, dz, xf, w1x, w1y, w1z, w1e2, b1g, w2gt, w2ge, W1f.T,
      b1f.reshape(1, _FH), W2f.T, b2f.reshape(1, _FH))
    return out


def kernel(xyz, features, W1g, W2g, W1f, b1f, W2f, b2f):
    b, n, _ = xyz.shape
    centroids = _fps_centroids(xyz)
    d2 = (jnp.sum(centroids ** 2, axis=-1)[:, :, None]
          + jnp.sum(xyz ** 2, axis=-1)[:, None, :]
          - 2.0 * jnp.einsum('bmd,bnd->bmn', centroids, xyz))
    nchunk = 8
    csz = _N // nchunk
    dc = (-d2).reshape(b, _M, nchunk, csz)
    cv, ci = jax.lax.top_k(dc, _K)
    cv = cv.reshape(b, _M, nchunk * _K)
    gidx = (ci + (jnp.arange(nchunk, dtype=jnp.int32) * csz)[None, None, :, None]
            ).reshape(b, _M, nchunk * _K)
    _, sel = jax.lax.top_k(cv, _K)
    group_idx = jnp.take_along_axis(gidx, sel, axis=2)
    idx2 = (group_idx
            + (jnp.arange(_B, dtype=jnp.int32) * _N)[:, None, None]
            ).reshape(_B * _M * _K // 128, 128)
    xf, gx, gy, gz = _sc_gather(features.reshape(_B * _N, _FD), xyz, idx2)

    def _coord(g, c):
        return (g.reshape(b, _M, _K) - centroids[:, :, c:c + 1]).reshape(b * _M, _K)

    dx = _coord(gx, 0)
    dy = _coord(gy, 1)
    dz = _coord(gz, 2)
    out = _mlps(dx, dy, dz, xf.reshape(b * _M, _K * _FD), W1g, W2g, W1f, b1f,
                W2f, b2f)
    return out.reshape(b, _M, _GH + _FH)


# speedup vs baseline: 11.2457x; 11.2457x over previous
import jax
import jax.numpy as jnp
from jax import lax
from jax.experimental import pallas as pl
from jax.experimental.pallas import tpu as pltpu
from jax.experimental.pallas import tpu_sc as plsc

_B, _N, _FD = 8, 4096, 128
_M, _K = 1024, 32
_GH, _FH = 256, 256

_ROWS = 512


def _fps_body(x_ref, y_ref, z_ref, cx_ref, cy_ref, cz_ref):
    X = x_ref[...]
    Y = y_ref[...]
    Z = z_ref[...]
    lane = jax.lax.broadcasted_iota(jnp.int32, (_B, _N), 1)
    big = jnp.int32(_N)

    mlane = jax.lax.broadcasted_iota(jnp.int32, (_B, _M), 1)

    first = lane == 0
    cx0 = jnp.sum(jnp.where(first, X, 0.0), axis=1, keepdims=True)
    cy0 = jnp.sum(jnp.where(first, Y, 0.0), axis=1, keepdims=True)
    cz0 = jnp.sum(jnp.where(first, Z, 0.0), axis=1, keepdims=True)
    zerosm = jnp.zeros((_B, _M), jnp.float32)
    cxs0 = jnp.where(mlane == 0, cx0, zerosm)
    cys0 = jnp.where(mlane == 0, cy0, zerosm)
    czs0 = jnp.where(mlane == 0, cz0, zerosm)

    dists0 = jnp.full((_B, _N), 1e10, dtype=jnp.float32)

    def body(i, state):
        dists, lx, ly, lz, cxs, cys, czs = state
        d = (X - lx) ** 2 + (Y - ly) ** 2 + (Z - lz) ** 2
        dists = jnp.minimum(dists, d)
        m = jnp.max(dists, axis=1, keepdims=True)
        jstar = jnp.min(jnp.where(dists == m, lane, big), axis=1, keepdims=True)
        sel = lane == jstar
        nx = jnp.sum(jnp.where(sel, X, 0.0), axis=1, keepdims=True)
        ny = jnp.sum(jnp.where(sel, Y, 0.0), axis=1, keepdims=True)
        nz = jnp.sum(jnp.where(sel, Z, 0.0), axis=1, keepdims=True)
        here = mlane == i
        cxs = jnp.where(here, nx, cxs)
        cys = jnp.where(here, ny, cys)
        czs = jnp.where(here, nz, czs)
        return dists, nx, ny, nz, cxs, cys, czs

    _, _, _, _, cxs, cys, czs = jax.lax.fori_loop(
        1, _M, body, (dists0, cx0, cy0, cz0, cxs0, cys0, czs0))
    cx_ref[...] = cxs
    cy_ref[...] = cys
    cz_ref[...] = czs


def _fps_centroids(xyz):
    x = xyz[:, :, 0]
    y = xyz[:, :, 1]
    z = xyz[:, :, 2]
    shp = jax.ShapeDtypeStruct((_B, _M), jnp.float32)
    cx, cy, cz = pl.pallas_call(
        _fps_body,
        out_shape=(shp, shp, shp),
    )(x, y, z)
    return jnp.stack([cx, cy, cz], axis=-1)


_NW = 32
_IPW = (_B * _M * _K) // _NW
_FCH = 256
_CCH = 2048


def _gather_body(feat_hbm, xp_hbm, yp_hbm, zp_hbm, idx_hbm,
                 xf_out, gx_out, gy_out, gz_out,
                 idx2_v, gx_v, gy_v, gz_v, rows_v, sem, semf):
    wid = lax.axis_index("s") * 2 + lax.axis_index("c")
    rows_per_w = _IPW // 128
    r0 = wid * rows_per_w

    def super_chunk(c, carry):
        rbase = r0 + c * 16
        pltpu.sync_copy(idx_hbm.at[pl.ds(rbase, 16)], idx2_v)

        def j_body(j, carry2):
            iv = idx2_v.at[j]
            cg1 = pltpu.async_copy(xp_hbm.at[iv], gx_v.at[j], sem)
            cg2 = pltpu.async_copy(yp_hbm.at[iv], gy_v.at[j], sem)
            cg3 = pltpu.async_copy(zp_hbm.at[iv], gz_v.at[j], sem)
            cf = pltpu.async_copy(feat_hbm.at[iv], rows_v, semf)
            cg1.wait()
            cg2.wait()
            cg3.wait()
            cf.wait()
            pltpu.sync_copy(rows_v, xf_out.at[pl.ds((rbase + j) * 128, 128)])
            return carry2

        lax.fori_loop(0, 16, j_body, 0)
        pltpu.sync_copy(gx_v, gx_out.at[pl.ds(rbase, 16)])
        pltpu.sync_copy(gy_v, gy_out.at[pl.ds(rbase, 16)])
        pltpu.sync_copy(gz_v, gz_out.at[pl.ds(rbase, 16)])
        return carry

    lax.fori_loop(0, rows_per_w // 16, super_chunk, 0)


def _sc_gather(features_flat, xyz, idx2):
    mesh = plsc.VectorSubcoreMesh(core_axis_name="c", subcore_axis_name="s")
    nrows = _B * _M * _K
    k = pl.kernel(
        _gather_body,
        mesh=mesh,
        out_type=(
            jax.ShapeDtypeStruct((nrows, _FD), jnp.float32),
            jax.ShapeDtypeStruct((nrows // 128, 128), jnp.float32),
            jax.ShapeDtypeStruct((nrows // 128, 128), jnp.float32),
            jax.ShapeDtypeStruct((nrows // 128, 128), jnp.float32),
        ),
        scratch_types=[
            pltpu.VMEM((16, 128), jnp.int32),
            pltpu.VMEM((16, 128), jnp.float32),
            pltpu.VMEM((16, 128), jnp.float32),
            pltpu.VMEM((16, 128), jnp.float32),
            pltpu.VMEM((128, _FD), jnp.float32),
            pltpu.SemaphoreType.DMA,
            pltpu.SemaphoreType.DMA,
        ],
    )
    xflat = xyz[:, :, 0].reshape(_B * _N)
    yflat = xyz[:, :, 1].reshape(_B * _N)
    zflat = xyz[:, :, 2].reshape(_B * _N)
    return k(features_flat, xflat, yflat, zflat, idx2)


_RPW = (_B * _M) // _NW
_CAP = _N + 16


def _lane_all(v, op):
    i = lax.iota(jnp.int32, 16)
    for sh in (8, 4, 2, 1):
        v = op(v, v.at[i ^ sh].get(mode="promise_in_bounds"))
    return v


def _prefix_sum16(m):
    i = lax.iota(jnp.int32, 16)
    s = m
    for sh in (1, 2, 4, 8):
        g = s.at[jnp.maximum(i - sh, 0)].get(mode="promise_in_bounds")
        s = s + jnp.where(i >= sh, g, 0)
    return s


def _lane_all(v, op):
    i = lax.iota(jnp.int32, 16)
    for sh in (8, 4, 2, 1):
        v = op(v, v.at[i ^ sh].get(mode="promise_in_bounds"))
    return v


def _topk_body(d2_hbm, out_hbm, row2_v, tmpi_v, out_v, sem):
    wid = lax.axis_index("s") * 2 + lax.axis_index("c")
    r0 = wid * _RPW
    iota16 = lax.iota(jnp.int32, 16)
    inf = jnp.float32(jnp.inf)
    inf_v = jnp.full((16,), inf)
    big_v = jnp.full((16,), jnp.int32(0x7FFFFFFF))

    pltpu.async_copy(d2_hbm.at[pl.ds(r0 * _N, _N)], row2_v.at[pl.ds(0, _N)],
                     sem)

    def row_body(r, carry):
        buf = r % 2
        base = pl.multiple_of(buf * _N, _N)
        pltpu.make_async_copy(
            d2_hbm.at[pl.ds((r0 + r) * _N, _N)],
            row2_v.at[pl.ds(base, _N)], sem).wait()

        @pl.when(r < _RPW - 1)
        def _prefetch():
            nbase = pl.multiple_of(((r + 1) % 2) * _N, _N)
            pltpu.async_copy(d2_hbm.at[pl.ds((r0 + r + 1) * _N, _N)],
                             row2_v.at[pl.ds(nbase, _N)], sem)

        def sbuild(g, S):
            def inner(j, m):
                s = pl.multiple_of(base + g * 256 + j * 16, 16)
                return jnp.minimum(m, row2_v[pl.ds(s, 16)])

            m = lax.fori_loop(0, 16, inner, inf_v)
            return jnp.where(iota16 == g, _lane_all(m, jnp.minimum), S)

        S = lax.fori_loop(0, 16, sbuild, inf_v)
        boff = ((r0 + r) // _M) * _N

        o0 = jnp.zeros((16,), jnp.int32)
        o1 = jnp.zeros((16,), jnp.int32)
        for k in range(_K):
            gm_v = _lane_all(S, jnp.minimum)
            g_v = _lane_all(jnp.where(S == gm_v, iota16, big_v), jnp.minimum)
            g = g_v[0]

            def scan(j, bp):
                s2 = pl.multiple_of(base + g * 256 + j * 16, 16)
                v = row2_v[pl.ds(s2, 16)]
                return jnp.minimum(
                    bp, jnp.where(v == gm_v, g * 256 + j * 16 + iota16, big_v))

            bp = lax.fori_loop(0, 16, scan, big_v)
            p_v = _lane_all(bp, jnp.minimum)
            pcol = p_v[0]
            if k < 16:
                o0 = jnp.where(iota16 == k, p_v + boff, o0)
            else:
                o1 = jnp.where(iota16 == (k - 16), p_v + boff, o1)
            voff = pl.multiple_of(base + (pcol >> 4) * 16, 16)
            vv = row2_v[pl.ds(voff, 16)]
            row2_v[pl.ds(voff, 16)] = jnp.where(iota16 == (p_v & 15), inf_v,
                                                vv)

            def refresh(j, m):
                s2 = pl.multiple_of(base + g * 256 + j * 16, 16)
                return jnp.minimum(m, row2_v[pl.ds(s2, 16)])

            m = lax.fori_loop(0, 16, refresh, inf_v)
            S = jnp.where(iota16 == g_v, _lane_all(m, jnp.minimum), S)

        out_v[pl.ds(0, 16)] = o0
        out_v[pl.ds(16, 16)] = o1
        pltpu.sync_copy(out_v, out_hbm.at[pl.ds((r0 + r) * _K, _K)])
        return carry

    lax.fori_loop(0, _RPW, row_body, 0)


def _sc_topk(d2flat):
    mesh = plsc.VectorSubcoreMesh(core_axis_name="c", subcore_axis_name="s")
    k = pl.kernel(
        _topk_body,
        mesh=mesh,
        out_type=jax.ShapeDtypeStruct((_B * _M * _K,), jnp.int32),
        scratch_types=[
            pltpu.VMEM((2 * _N,), jnp.float32),
            pltpu.VMEM((16,), jnp.int32),
            pltpu.VMEM((_K,), jnp.int32),
            pltpu.SemaphoreType.DMA,
        ],
    )
    return k(d2flat)


def _mlp_block(dx_ref, dy_ref, dz_ref, xf_ref, w1x_ref, w1y_ref, w1z_ref,
               w1e2_ref, b1g_ref, w2gt_ref, w2ge_ref, w1ft_ref, b1f_ref,
               w2ft_ref, b2f_ref, out_ref):
    dx = dx_ref[...]
    dy = dy_ref[...]
    dz = dz_ref[...]
    e2 = -0.5 * (dx * dx + dy * dy + dz * dz)
    h = (jnp.dot(dx, w1x_ref[...], preferred_element_type=jnp.float32)
         + jnp.dot(dy, w1y_ref[...], preferred_element_type=jnp.float32)
         + jnp.dot(dz, w1z_ref[...], preferred_element_type=jnp.float32)
         + jnp.dot(e2, w1e2_ref[...], preferred_element_type=jnp.float32)
         + b1g_ref[...])
    e2h = -0.5 * jnp.sum(h * h, axis=1, keepdims=True)
    xgeo = (jnp.dot(h, w2gt_ref[...], preferred_element_type=jnp.float32)
            - w2ge_ref[0, :][None, :] + e2h * w2ge_ref[1, :][None, :])
    hf = jnp.maximum(
        jnp.dot(xf_ref[...], w1ft_ref[...], preferred_element_type=jnp.float32)
        + b1f_ref[...], 0.0)
    xfeat = jnp.dot(hf, w2ft_ref[...], preferred_element_type=jnp.float32) + b2f_ref[...]
    out_ref[:, :_GH] = xgeo
    out_ref[:, _GH:] = xfeat


def _mlps(dx, dy, dz, xf, W1g, W2g, W1f, b1f, W2f, b2f):
    nrows = dx.shape[0]
    grid = nrows // _ROWS
    w1gt = W1g.T
    w1x = w1gt[0::5]
    w1y = w1gt[1::5]
    w1z = w1gt[2::5]
    w1e2 = w1gt[4::5]
    b1g = -jnp.sum(w1gt[3::5], axis=0, keepdims=True)
    w2gt = W2g[:, :_GH].T
    w2ge = W2g[:, _GH:].T
    kspec = pl.BlockSpec((_K, _GH), lambda i: (0, 0))
    bspec = pl.BlockSpec((1, _FH), lambda i: (0, 0))
    rkspec = pl.BlockSpec((_ROWS, _K), lambda i: (i, 0))
    out = pl.pallas_call(
        _mlp_block,
        grid=(grid,),
        in_specs=[
            rkspec, rkspec, rkspec,
            pl.BlockSpec((_ROWS, _K * _FD), lambda i: (i, 0)),
            kspec, kspec, kspec, kspec,
            bspec,
            pl.BlockSpec((_GH, _GH), lambda i: (0, 0)),
            pl.BlockSpec((2, _GH), lambda i: (0, 0)),
            pl.BlockSpec((_K * _FD, _FH), lambda i: (0, 0)),
            bspec,
            pl.BlockSpec((_FH, _FH), lambda i: (0, 0)),
            bspec,
        ],
        out_specs=pl.BlockSpec((_ROWS, _GH + _FH), lambda i: (i, 0)),
        out_shape=jax.ShapeDtypeStruct((nrows, _GH + _FH), jnp.float32),
    )(dx, dy, dz, xf, w1x, w1y, w1z, w1e2, b1g, w2gt, w2ge, W1f.T,
      b1f.reshape(1, _FH), W2f.T, b2f.reshape(1, _FH))
    return out


def kernel(xyz, features, W1g, W2g, W1f, b1f, W2f, b2f):
    b, n, _ = xyz.shape
    centroids = _fps_centroids(xyz)
    d2 = (jnp.sum(centroids ** 2, axis=-1)[:, :, None]
          + jnp.sum(xyz ** 2, axis=-1)[:, None, :]
          - 2.0 * jnp.einsum('bmd,bnd->bmn', centroids, xyz))
    idx2 = _sc_topk(d2.reshape(_B * _M * _N)).reshape(_B * _M * _K // 128, 128)
    xf, gx, gy, gz = _sc_gather(features.reshape(_B * _N, _FD), xyz, idx2)

    def _coord(g, c):
        return (g.reshape(b, _M, _K) - centroids[:, :, c:c + 1]).reshape(b * _M, _K)

    dx = _coord(gx, 0)
    dy = _coord(gy, 1)
    dz = _coord(gz, 2)
    out = _mlps(dx, dy, dz, xf.reshape(b * _M, _K * _FD), W1g, W2g, W1f, b1f,
                W2f, b2f)
    return out.reshape(b, _M, _GH + _FH)


# speedup vs baseline: 13.6059x; 1.2099x over previous
import jax
import jax.numpy as jnp
from jax import lax
from jax.experimental import pallas as pl
from jax.experimental.pallas import tpu as pltpu
from jax.experimental.pallas import tpu_sc as plsc

_B, _N, _FD = 8, 4096, 128
_M, _K = 1024, 32
_GH, _FH = 256, 256

_ROWS = 512


def _fps_body(x_ref, y_ref, z_ref, cx_ref, cy_ref, cz_ref):
    X = x_ref[...]
    Y = y_ref[...]
    Z = z_ref[...]
    lane = jax.lax.broadcasted_iota(jnp.int32, (_B, _N), 1)
    big = jnp.int32(_N)

    mlane = jax.lax.broadcasted_iota(jnp.int32, (_B, _M), 1)

    first = lane == 0
    cx0 = jnp.sum(jnp.where(first, X, 0.0), axis=1, keepdims=True)
    cy0 = jnp.sum(jnp.where(first, Y, 0.0), axis=1, keepdims=True)
    cz0 = jnp.sum(jnp.where(first, Z, 0.0), axis=1, keepdims=True)
    zerosm = jnp.zeros((_B, _M), jnp.float32)
    cxs0 = jnp.where(mlane == 0, cx0, zerosm)
    cys0 = jnp.where(mlane == 0, cy0, zerosm)
    czs0 = jnp.where(mlane == 0, cz0, zerosm)

    dists0 = jnp.full((_B, _N), 1e10, dtype=jnp.float32)

    def body(i, state):
        dists, lx, ly, lz, cxs, cys, czs = state
        d = (X - lx) ** 2 + (Y - ly) ** 2 + (Z - lz) ** 2
        dists = jnp.minimum(dists, d)
        m = jnp.max(dists, axis=1, keepdims=True)
        jstar = jnp.min(jnp.where(dists == m, lane, big), axis=1, keepdims=True)
        sel = lane == jstar
        nx = jnp.sum(jnp.where(sel, X, 0.0), axis=1, keepdims=True)
        ny = jnp.sum(jnp.where(sel, Y, 0.0), axis=1, keepdims=True)
        nz = jnp.sum(jnp.where(sel, Z, 0.0), axis=1, keepdims=True)
        here = mlane == i
        cxs = jnp.where(here, nx, cxs)
        cys = jnp.where(here, ny, cys)
        czs = jnp.where(here, nz, czs)
        return dists, nx, ny, nz, cxs, cys, czs

    _, _, _, _, cxs, cys, czs = jax.lax.fori_loop(
        1, _M, body, (dists0, cx0, cy0, cz0, cxs0, cys0, czs0))
    cx_ref[...] = cxs
    cy_ref[...] = cys
    cz_ref[...] = czs


def _fps_centroids(xyz):
    x = xyz[:, :, 0]
    y = xyz[:, :, 1]
    z = xyz[:, :, 2]
    shp = jax.ShapeDtypeStruct((_B, _M), jnp.float32)
    cx, cy, cz = pl.pallas_call(
        _fps_body,
        out_shape=(shp, shp, shp),
    )(x, y, z)
    return jnp.stack([cx, cy, cz], axis=-1)


_NW = 32
_IPW = (_B * _M * _K) // _NW
_FCH = 256
_CCH = 2048


def _gather_body(feat_hbm, xp_hbm, yp_hbm, zp_hbm, idx_hbm,
                 xf_out, gx_out, gy_out, gz_out,
                 idx2_v, gx_v, gy_v, gz_v, rows_v, sem, semf):
    wid = lax.axis_index("s") * 2 + lax.axis_index("c")
    rows_per_w = _IPW // 128
    r0 = wid * rows_per_w

    def super_chunk(c, carry):
        rbase = r0 + c * 16
        pltpu.sync_copy(idx_hbm.at[pl.ds(rbase, 16)], idx2_v)

        def j_body(j, carry2):
            iv = idx2_v.at[j]
            cg1 = pltpu.async_copy(xp_hbm.at[iv], gx_v.at[j], sem)
            cg2 = pltpu.async_copy(yp_hbm.at[iv], gy_v.at[j], sem)
            cg3 = pltpu.async_copy(zp_hbm.at[iv], gz_v.at[j], sem)
            cf = pltpu.async_copy(feat_hbm.at[iv], rows_v, semf)
            cg1.wait()
            cg2.wait()
            cg3.wait()
            cf.wait()
            pltpu.sync_copy(rows_v, xf_out.at[pl.ds((rbase + j) * 128, 128)])
            return carry2

        lax.fori_loop(0, 16, j_body, 0)
        pltpu.sync_copy(gx_v, gx_out.at[pl.ds(rbase, 16)])
        pltpu.sync_copy(gy_v, gy_out.at[pl.ds(rbase, 16)])
        pltpu.sync_copy(gz_v, gz_out.at[pl.ds(rbase, 16)])
        return carry

    lax.fori_loop(0, rows_per_w // 16, super_chunk, 0)


def _sc_gather(features_flat, xyz, idx2):
    mesh = plsc.VectorSubcoreMesh(core_axis_name="c", subcore_axis_name="s")
    nrows = _B * _M * _K
    k = pl.kernel(
        _gather_body,
        mesh=mesh,
        out_type=(
            jax.ShapeDtypeStruct((nrows, _FD), jnp.float32),
            jax.ShapeDtypeStruct((nrows // 128, 128), jnp.float32),
            jax.ShapeDtypeStruct((nrows // 128, 128), jnp.float32),
            jax.ShapeDtypeStruct((nrows // 128, 128), jnp.float32),
        ),
        scratch_types=[
            pltpu.VMEM((16, 128), jnp.int32),
            pltpu.VMEM((16, 128), jnp.float32),
            pltpu.VMEM((16, 128), jnp.float32),
            pltpu.VMEM((16, 128), jnp.float32),
            pltpu.VMEM((128, _FD), jnp.float32),
            pltpu.SemaphoreType.DMA,
            pltpu.SemaphoreType.DMA,
        ],
    )
    xflat = xyz[:, :, 0].reshape(_B * _N)
    yflat = xyz[:, :, 1].reshape(_B * _N)
    zflat = xyz[:, :, 2].reshape(_B * _N)
    return k(features_flat, xflat, yflat, zflat, idx2)


_RPW = (_B * _M) // _NW
_CAP = _N + 16


def _lane_all(v, op):
    i = lax.iota(jnp.int32, 16)
    for sh in (8, 4, 2, 1):
        v = op(v, v.at[i ^ sh].get(mode="promise_in_bounds"))
    return v


def _prefix_sum16(m):
    i = lax.iota(jnp.int32, 16)
    s = m
    for sh in (1, 2, 4, 8):
        g = s.at[jnp.maximum(i - sh, 0)].get(mode="promise_in_bounds")
        s = s + jnp.where(i >= sh, g, 0)
    return s


def _lane_all(v, op):
    i = lax.iota(jnp.int32, 16)
    for sh in (8, 4, 2, 1):
        v = op(v, v.at[i ^ sh].get(mode="promise_in_bounds"))
    return v


def _topk_body(d2_hbm, out_hbm, row2_v, tmpi_v, out_v, sem):
    wid = lax.axis_index("s") * 2 + lax.axis_index("c")
    r0 = wid * _RPW
    iota16 = lax.iota(jnp.int32, 16)
    inf = jnp.float32(jnp.inf)
    inf_v = jnp.full((16,), inf)
    big_v = jnp.full((16,), jnp.int32(0x7FFFFFFF))

    pltpu.async_copy(d2_hbm.at[pl.ds(r0 * _N, _N)], row2_v.at[pl.ds(0, _N)],
                     sem)

    def row_body(r, carry):
        buf = r % 2
        base = pl.multiple_of(buf * _N, _N)
        pltpu.make_async_copy(
            d2_hbm.at[pl.ds((r0 + r) * _N, _N)],
            row2_v.at[pl.ds(base, _N)], sem).wait()

        @pl.when(r < _RPW - 1)
        def _prefetch():
            nbase = pl.multiple_of(((r + 1) % 2) * _N, _N)
            pltpu.async_copy(d2_hbm.at[pl.ds((r0 + r + 1) * _N, _N)],
                             row2_v.at[pl.ds(nbase, _N)], sem)

        def sbuild(g, S):
            def inner(j, m):
                s = pl.multiple_of(base + g * 256 + j * 16, 16)
                return jnp.minimum(m, row2_v[pl.ds(s, 16)])

            m = lax.fori_loop(0, 16, inner, inf_v)
            return jnp.where(iota16 == g, _lane_all(m, jnp.minimum), S)

        S = lax.fori_loop(0, 16, sbuild, inf_v)
        boff = ((r0 + r) // _M) * _N

        o0 = jnp.zeros((16,), jnp.int32)
        o1 = jnp.zeros((16,), jnp.int32)
        for k in range(_K):
            gm_v = _lane_all(S, jnp.minimum)
            g_v = _lane_all(jnp.where(S == gm_v, iota16, big_v), jnp.minimum)
            g = g_v[0]

            def scan(j, c3):
                bp, ng, mx = c3
                s2 = pl.multiple_of(base + g * 256 + j * 16, 16)
                v = row2_v[pl.ds(s2, 16)]
                eq = v == gm_v
                bp = jnp.minimum(
                    bp, jnp.where(eq, g * 256 + j * 16 + iota16, big_v))
                ng = ng + jnp.where(eq, jnp.int32(1), jnp.int32(0))
                mx = jnp.minimum(mx, jnp.where(eq, inf_v, v))
                return bp, ng, mx

            bp, ng, mx = lax.fori_loop(
                0, 16, scan, (big_v, jnp.zeros((16,), jnp.int32), inf_v))
            p_v = _lane_all(bp, jnp.minimum)
            pcol = p_v[0]
            if k < 16:
                o0 = jnp.where(iota16 == k, p_v + boff, o0)
            else:
                o1 = jnp.where(iota16 == (k - 16), p_v + boff, o1)
            voff = pl.multiple_of(base + (pcol >> 4) * 16, 16)
            vv = row2_v[pl.ds(voff, 16)]
            row2_v[pl.ds(voff, 16)] = jnp.where(iota16 == (p_v & 15), inf_v,
                                                vv)
            ngt = _lane_all(ng, jnp.add)
            mexcl = _lane_all(mx, jnp.minimum)
            newmin = jnp.where(ngt > 1, gm_v, mexcl)
            S = jnp.where(iota16 == g_v, newmin, S)

        out_v[pl.ds(0, 16)] = o0
        out_v[pl.ds(16, 16)] = o1
        pltpu.sync_copy(out_v, out_hbm.at[pl.ds((r0 + r) * _K, _K)])
        return carry

    lax.fori_loop(0, _RPW, row_body, 0)


def _sc_topk(d2flat):
    mesh = plsc.VectorSubcoreMesh(core_axis_name="c", subcore_axis_name="s")
    k = pl.kernel(
        _topk_body,
        mesh=mesh,
        out_type=jax.ShapeDtypeStruct((_B * _M * _K,), jnp.int32),
        scratch_types=[
            pltpu.VMEM((2 * _N,), jnp.float32),
            pltpu.VMEM((16,), jnp.int32),
            pltpu.VMEM((_K,), jnp.int32),
            pltpu.SemaphoreType.DMA,
        ],
    )
    return k(d2flat)


def _mlp_block(dx_ref, dy_ref, dz_ref, xf_ref, w1x_ref, w1y_ref, w1z_ref,
               w1e2_ref, b1g_ref, w2gt_ref, w2ge_ref, w1ft_ref, b1f_ref,
               w2ft_ref, b2f_ref, out_ref):
    dx = dx_ref[...]
    dy = dy_ref[...]
    dz = dz_ref[...]
    e2 = -0.5 * (dx * dx + dy * dy + dz * dz)
    h = (jnp.dot(dx, w1x_ref[...], preferred_element_type=jnp.float32)
         + jnp.dot(dy, w1y_ref[...], preferred_element_type=jnp.float32)
         + jnp.dot(dz, w1z_ref[...], preferred_element_type=jnp.float32)
         + jnp.dot(e2, w1e2_ref[...], preferred_element_type=jnp.float32)
         + b1g_ref[...])
    e2h = -0.5 * jnp.sum(h * h, axis=1, keepdims=True)
    xgeo = (jnp.dot(h, w2gt_ref[...], preferred_element_type=jnp.float32)
            - w2ge_ref[0, :][None, :] + e2h * w2ge_ref[1, :][None, :])
    hf = jnp.maximum(
        jnp.dot(xf_ref[...], w1ft_ref[...], preferred_element_type=jnp.float32)
        + b1f_ref[...], 0.0)
    xfeat = jnp.dot(hf, w2ft_ref[...], preferred_element_type=jnp.float32) + b2f_ref[...]
    out_ref[:, :_GH] = xgeo
    out_ref[:, _GH:] = xfeat


def _mlps(dx, dy, dz, xf, W1g, W2g, W1f, b1f, W2f, b2f):
    nrows = dx.shape[0]
    grid = nrows // _ROWS
    w1gt = W1g.T
    w1x = w1gt[0::5]
    w1y = w1gt[1::5]
    w1z = w1gt[2::5]
    w1e2 = w1gt[4::5]
    b1g = -jnp.sum(w1gt[3::5], axis=0, keepdims=True)
    w2gt = W2g[:, :_GH].T
    w2ge = W2g[:, _GH:].T
    kspec = pl.BlockSpec((_K, _GH), lambda i: (0, 0))
    bspec = pl.BlockSpec((1, _FH), lambda i: (0, 0))
    rkspec = pl.BlockSpec((_ROWS, _K), lambda i: (i, 0))
    out = pl.pallas_call(
        _mlp_block,
        grid=(grid,),
        in_specs=[
            rkspec, rkspec, rkspec,
            pl.BlockSpec((_ROWS, _K * _FD), lambda i: (i, 0)),
            kspec, kspec, kspec, kspec,
            bspec,
            pl.BlockSpec((_GH, _GH), lambda i: (0, 0)),
            pl.BlockSpec((2, _GH), lambda i: (0, 0)),
            pl.BlockSpec((_K * _FD, _FH), lambda i: (0, 0)),
            bspec,
            pl.BlockSpec((_FH, _FH), lambda i: (0, 0)),
            bspec,
        ],
        out_specs=pl.BlockSpec((_ROWS, _GH + _FH), lambda i: (i, 0)),
        out_shape=jax.ShapeDtypeStruct((nrows, _GH + _FH), jnp.float32),
    )(dx, dy, dz, xf, w1x, w1y, w1z, w1e2, b1g, w2gt, w2ge, W1f.T,
      b1f.reshape(1, _FH), W2f.T, b2f.reshape(1, _FH))
    return out


def kernel(xyz, features, W1g, W2g, W1f, b1f, W2f, b2f):
    b, n, _ = xyz.shape
    centroids = _fps_centroids(xyz)
    d2 = (jnp.sum(centroids ** 2, axis=-1)[:, :, None]
          + jnp.sum(xyz ** 2, axis=-1)[:, None, :]
          - 2.0 * jnp.einsum('bmd,bnd->bmn', centroids, xyz))
    idx2 = _sc_topk(d2.reshape(_B * _M * _N)).reshape(_B * _M * _K // 128, 128)
    xf, gx, gy, gz = _sc_gather(features.reshape(_B * _N, _FD), xyz, idx2)

    def _coord(g, c):
        return (g.reshape(b, _M, _K) - centroids[:, :, c:c + 1]).reshape(b * _M, _K)

    dx = _coord(gx, 0)
    dy = _coord(gy, 1)
    dz = _coord(gz, 2)
    out = _mlps(dx, dy, dz, xf.reshape(b * _M, _K * _FD), W1g, W2g, W1f, b1f,
                W2f, b2f)
    return out.reshape(b, _M, _GH + _FH)


# speedup vs baseline: 16.7606x; 1.2319x over previous
import jax
import jax.numpy as jnp
from jax import lax
from jax.experimental import pallas as pl
from jax.experimental.pallas import tpu as pltpu
from jax.experimental.pallas import tpu_sc as plsc

_B, _N, _FD = 8, 4096, 128
_M, _K = 1024, 32
_GH, _FH = 256, 256

_ROWS = 512


def _fps_body(x_ref, y_ref, z_ref, cx_ref, cy_ref, cz_ref):
    X = x_ref[...]
    Y = y_ref[...]
    Z = z_ref[...]
    lane = jax.lax.broadcasted_iota(jnp.int32, (_B, _N), 1)
    big = jnp.int32(_N)

    mlane = jax.lax.broadcasted_iota(jnp.int32, (_B, _M), 1)

    first = lane == 0
    cx0 = jnp.sum(jnp.where(first, X, 0.0), axis=1, keepdims=True)
    cy0 = jnp.sum(jnp.where(first, Y, 0.0), axis=1, keepdims=True)
    cz0 = jnp.sum(jnp.where(first, Z, 0.0), axis=1, keepdims=True)
    zerosm = jnp.zeros((_B, _M), jnp.float32)
    cxs0 = jnp.where(mlane == 0, cx0, zerosm)
    cys0 = jnp.where(mlane == 0, cy0, zerosm)
    czs0 = jnp.where(mlane == 0, cz0, zerosm)

    dists0 = jnp.full((_B, _N), 1e10, dtype=jnp.float32)

    def body(i, state):
        dists, lx, ly, lz, cxs, cys, czs = state
        d = (X - lx) ** 2 + (Y - ly) ** 2 + (Z - lz) ** 2
        dists = jnp.minimum(dists, d)
        m = jnp.max(dists, axis=1, keepdims=True)
        jstar = jnp.min(jnp.where(dists == m, lane, big), axis=1, keepdims=True)
        sel = lane == jstar
        nx = jnp.sum(jnp.where(sel, X, 0.0), axis=1, keepdims=True)
        ny = jnp.sum(jnp.where(sel, Y, 0.0), axis=1, keepdims=True)
        nz = jnp.sum(jnp.where(sel, Z, 0.0), axis=1, keepdims=True)
        here = mlane == i
        cxs = jnp.where(here, nx, cxs)
        cys = jnp.where(here, ny, cys)
        czs = jnp.where(here, nz, czs)
        return dists, nx, ny, nz, cxs, cys, czs

    _, _, _, _, cxs, cys, czs = jax.lax.fori_loop(
        1, _M, body, (dists0, cx0, cy0, cz0, cxs0, cys0, czs0))
    cx_ref[...] = cxs
    cy_ref[...] = cys
    cz_ref[...] = czs


def _fps_centroids(xyz):
    x = xyz[:, :, 0]
    y = xyz[:, :, 1]
    z = xyz[:, :, 2]
    shp = jax.ShapeDtypeStruct((_B, _M), jnp.float32)
    cx, cy, cz = pl.pallas_call(
        _fps_body,
        out_shape=(shp, shp, shp),
    )(x, y, z)
    return jnp.stack([cx, cy, cz], axis=-1)


_NW = 32
_IPW = (_B * _M * _K) // _NW
_FCH = 256
_CCH = 2048


def _gather_body(feat_hbm, xp_hbm, yp_hbm, zp_hbm, idx_hbm,
                 xf_out, gx_out, gy_out, gz_out,
                 idx2_v, gx_v, gy_v, gz_v, rows_v, sem, semf):
    wid = lax.axis_index("s") * 2 + lax.axis_index("c")
    rows_per_w = _IPW // 128
    r0 = wid * rows_per_w

    def super_chunk(c, carry):
        rbase = r0 + c * 16
        pltpu.sync_copy(idx_hbm.at[pl.ds(rbase, 16)], idx2_v)

        def j_body(j, carry2):
            iv = idx2_v.at[j]
            cg1 = pltpu.async_copy(xp_hbm.at[iv], gx_v.at[j], sem)
            cg2 = pltpu.async_copy(yp_hbm.at[iv], gy_v.at[j], sem)
            cg3 = pltpu.async_copy(zp_hbm.at[iv], gz_v.at[j], sem)
            cf = pltpu.async_copy(feat_hbm.at[iv], rows_v, semf)
            cg1.wait()
            cg2.wait()
            cg3.wait()
            cf.wait()
            pltpu.sync_copy(rows_v, xf_out.at[pl.ds((rbase + j) * 128, 128)])
            return carry2

        lax.fori_loop(0, 16, j_body, 0)
        pltpu.sync_copy(gx_v, gx_out.at[pl.ds(rbase, 16)])
        pltpu.sync_copy(gy_v, gy_out.at[pl.ds(rbase, 16)])
        pltpu.sync_copy(gz_v, gz_out.at[pl.ds(rbase, 16)])
        return carry

    lax.fori_loop(0, rows_per_w // 16, super_chunk, 0)


def _sc_gather(features_flat, xyz, idx2):
    mesh = plsc.VectorSubcoreMesh(core_axis_name="c", subcore_axis_name="s")
    nrows = _B * _M * _K
    k = pl.kernel(
        _gather_body,
        mesh=mesh,
        out_type=(
            jax.ShapeDtypeStruct((nrows, _FD), jnp.float32),
            jax.ShapeDtypeStruct((nrows // 128, 128), jnp.float32),
            jax.ShapeDtypeStruct((nrows // 128, 128), jnp.float32),
            jax.ShapeDtypeStruct((nrows // 128, 128), jnp.float32),
        ),
        scratch_types=[
            pltpu.VMEM((16, 128), jnp.int32),
            pltpu.VMEM((16, 128), jnp.float32),
            pltpu.VMEM((16, 128), jnp.float32),
            pltpu.VMEM((16, 128), jnp.float32),
            pltpu.VMEM((128, _FD), jnp.float32),
            pltpu.SemaphoreType.DMA,
            pltpu.SemaphoreType.DMA,
        ],
    )
    xflat = xyz[:, :, 0].reshape(_B * _N)
    yflat = xyz[:, :, 1].reshape(_B * _N)
    zflat = xyz[:, :, 2].reshape(_B * _N)
    return k(features_flat, xflat, yflat, zflat, idx2)


_RPW = (_B * _M) // _NW
_CAP = _N + 16


def _lane_all(v, op):
    i = lax.iota(jnp.int32, 16)
    for sh in (8, 4, 2, 1):
        v = op(v, v.at[i ^ sh].get(mode="promise_in_bounds"))
    return v


def _prefix_sum16(m):
    i = lax.iota(jnp.int32, 16)
    s = m
    for sh in (1, 2, 4, 8):
        g = s.at[jnp.maximum(i - sh, 0)].get(mode="promise_in_bounds")
        s = s + jnp.where(i >= sh, g, 0)
    return s


def _lane_all(v, op):
    i = lax.iota(jnp.int32, 16)
    for sh in (8, 4, 2, 1):
        v = op(v, v.at[i ^ sh].get(mode="promise_in_bounds"))
    return v


def _topk_body(d2_hbm, out_hbm, row2_v, tmpi_v, out_v, sem):
    wid = lax.axis_index("s") * 2 + lax.axis_index("c")
    r0 = wid * _RPW
    iota16 = lax.iota(jnp.int32, 16)
    inf = jnp.float32(jnp.inf)
    inf_v = jnp.full((16,), inf)
    big_v = jnp.full((16,), jnp.int32(0x7FFFFFFF))

    pltpu.async_copy(d2_hbm.at[pl.ds(r0 * _N, _N)], row2_v.at[pl.ds(0, _N)],
                     sem)
    pltpu.async_copy(d2_hbm.at[pl.ds((r0 + 1) * _N, _N)],
                     row2_v.at[pl.ds(_N, _N)], sem)

    def row_body(r2, carry):
        buf = r2 % 2
        ra = r0 + 2 * r2
        base_a = pl.multiple_of(buf * 2 * _N, _N)
        base_b = pl.multiple_of(buf * 2 * _N + _N, _N)
        pltpu.make_async_copy(
            d2_hbm.at[pl.ds(ra * _N, _N)],
            row2_v.at[pl.ds(base_a, _N)], sem).wait()
        pltpu.make_async_copy(
            d2_hbm.at[pl.ds((ra + 1) * _N, _N)],
            row2_v.at[pl.ds(base_b, _N)], sem).wait()

        @pl.when(r2 < _RPW // 2 - 1)
        def _prefetch():
            nb = pl.multiple_of(((r2 + 1) % 2) * 2 * _N, _N)
            pltpu.async_copy(d2_hbm.at[pl.ds((ra + 2) * _N, _N)],
                             row2_v.at[pl.ds(nb, _N)], sem)
            pltpu.async_copy(d2_hbm.at[pl.ds((ra + 3) * _N, _N)],
                             row2_v.at[pl.ds(nb + _N, _N)], sem)

        def sbuild(g, Ss):
            Sa, Sb = Ss

            def inner(j, ms):
                ma, mb = ms
                sa = pl.multiple_of(base_a + g * 256 + j * 16, 16)
                sb = pl.multiple_of(base_b + g * 256 + j * 16, 16)
                return (jnp.minimum(ma, row2_v[pl.ds(sa, 16)]),
                        jnp.minimum(mb, row2_v[pl.ds(sb, 16)]))

            ma, mb = lax.fori_loop(0, 16, inner, (inf_v, inf_v))
            Sa = jnp.where(iota16 == g, _lane_all(ma, jnp.minimum), Sa)
            Sb = jnp.where(iota16 == g, _lane_all(mb, jnp.minimum), Sb)
            return Sa, Sb

        Sa, Sb = lax.fori_loop(0, 16, sbuild, (inf_v, inf_v))
        boff_a = (ra // _M) * _N
        boff_b = ((ra + 1) // _M) * _N

        o0a = jnp.zeros((16,), jnp.int32)
        o1a = jnp.zeros((16,), jnp.int32)
        o0b = jnp.zeros((16,), jnp.int32)
        o1b = jnp.zeros((16,), jnp.int32)
        for k in range(_K):
            gma_v = _lane_all(Sa, jnp.minimum)
            gmb_v = _lane_all(Sb, jnp.minimum)
            ga_v = _lane_all(jnp.where(Sa == gma_v, iota16, big_v),
                             jnp.minimum)
            gb_v = _lane_all(jnp.where(Sb == gmb_v, iota16, big_v),
                             jnp.minimum)
            ga = ga_v[0]
            gb = gb_v[0]

            def scan(j, c6):
                bpa, nga, mxa, bpb, ngb, mxb = c6
                sa = pl.multiple_of(base_a + ga * 256 + j * 16, 16)
                sb = pl.multiple_of(base_b + gb * 256 + j * 16, 16)
                va = row2_v[pl.ds(sa, 16)]
                vb = row2_v[pl.ds(sb, 16)]
                eqa = va == gma_v
                eqb = vb == gmb_v
                bpa = jnp.minimum(
                    bpa, jnp.where(eqa, ga * 256 + j * 16 + iota16, big_v))
                bpb = jnp.minimum(
                    bpb, jnp.where(eqb, gb * 256 + j * 16 + iota16, big_v))
                nga = nga + jnp.where(eqa, jnp.int32(1), jnp.int32(0))
                ngb = ngb + jnp.where(eqb, jnp.int32(1), jnp.int32(0))
                mxa = jnp.minimum(mxa, jnp.where(eqa, inf_v, va))
                mxb = jnp.minimum(mxb, jnp.where(eqb, inf_v, vb))
                return bpa, nga, mxa, bpb, ngb, mxb

            zi = jnp.zeros((16,), jnp.int32)
            bpa, nga, mxa, bpb, ngb, mxb = lax.fori_loop(
                0, 16, scan, (big_v, zi, inf_v, big_v, zi, inf_v))
            pa_v = _lane_all(bpa, jnp.minimum)
            pb_v = _lane_all(bpb, jnp.minimum)
            pca = pa_v[0]
            pcb = pb_v[0]
            if k < 16:
                o0a = jnp.where(iota16 == k, pa_v + boff_a, o0a)
                o0b = jnp.where(iota16 == k, pb_v + boff_b, o0b)
            else:
                o1a = jnp.where(iota16 == (k - 16), pa_v + boff_a, o1a)
                o1b = jnp.where(iota16 == (k - 16), pb_v + boff_b, o1b)
            voa = pl.multiple_of(base_a + (pca >> 4) * 16, 16)
            vob = pl.multiple_of(base_b + (pcb >> 4) * 16, 16)
            vva = row2_v[pl.ds(voa, 16)]
            vvb = row2_v[pl.ds(vob, 16)]
            row2_v[pl.ds(voa, 16)] = jnp.where(iota16 == (pa_v & 15), inf_v,
                                               vva)
            row2_v[pl.ds(vob, 16)] = jnp.where(iota16 == (pb_v & 15), inf_v,
                                               vvb)
            ngta = _lane_all(nga, jnp.add)
            ngtb = _lane_all(ngb, jnp.add)
            mea = _lane_all(mxa, jnp.minimum)
            meb = _lane_all(mxb, jnp.minimum)
            Sa = jnp.where(iota16 == ga_v,
                           jnp.where(ngta > 1, gma_v, mea), Sa)
            Sb = jnp.where(iota16 == gb_v,
                           jnp.where(ngtb > 1, gmb_v, meb), Sb)

        out_v[pl.ds(0, 16)] = o0a
        out_v[pl.ds(16, 16)] = o1a
        out_v[pl.ds(32, 16)] = o0b
        out_v[pl.ds(48, 16)] = o1b
        pltpu.sync_copy(out_v, out_hbm.at[pl.ds(ra * _K, 2 * _K)])
        return carry

    lax.fori_loop(0, _RPW // 2, row_body, 0)


def _sc_topk(d2flat):
    mesh = plsc.VectorSubcoreMesh(core_axis_name="c", subcore_axis_name="s")
    k = pl.kernel(
        _topk_body,
        mesh=mesh,
        out_type=jax.ShapeDtypeStruct((_B * _M * _K,), jnp.int32),
        scratch_types=[
            pltpu.VMEM((4 * _N,), jnp.float32),
            pltpu.VMEM((16,), jnp.int32),
            pltpu.VMEM((2 * _K,), jnp.int32),
            pltpu.SemaphoreType.DMA,
        ],
    )
    return k(d2flat)


def _mlp_block(dx_ref, dy_ref, dz_ref, xf_ref, w1x_ref, w1y_ref, w1z_ref,
               w1e2_ref, b1g_ref, w2gt_ref, w2ge_ref, w1ft_ref, b1f_ref,
               w2ft_ref, b2f_ref, out_ref):
    dx = dx_ref[...]
    dy = dy_ref[...]
    dz = dz_ref[...]
    e2 = -0.5 * (dx * dx + dy * dy + dz * dz)
    h = (jnp.dot(dx, w1x_ref[...], preferred_element_type=jnp.float32)
         + jnp.dot(dy, w1y_ref[...], preferred_element_type=jnp.float32)
         + jnp.dot(dz, w1z_ref[...], preferred_element_type=jnp.float32)
         + jnp.dot(e2, w1e2_ref[...], preferred_element_type=jnp.float32)
         + b1g_ref[...])
    e2h = -0.5 * jnp.sum(h * h, axis=1, keepdims=True)
    xgeo = (jnp.dot(h, w2gt_ref[...], preferred_element_type=jnp.float32)
            - w2ge_ref[0, :][None, :] + e2h * w2ge_ref[1, :][None, :])
    hf = jnp.maximum(
        jnp.dot(xf_ref[...], w1ft_ref[...], preferred_element_type=jnp.float32)
        + b1f_ref[...], 0.0)
    xfeat = jnp.dot(hf, w2ft_ref[...], preferred_element_type=jnp.float32) + b2f_ref[...]
    out_ref[:, :_GH] = xgeo
    out_ref[:, _GH:] = xfeat


def _mlps(dx, dy, dz, xf, W1g, W2g, W1f, b1f, W2f, b2f):
    nrows = dx.shape[0]
    grid = nrows // _ROWS
    w1gt = W1g.T
    w1x = w1gt[0::5]
    w1y = w1gt[1::5]
    w1z = w1gt[2::5]
    w1e2 = w1gt[4::5]
    b1g = -jnp.sum(w1gt[3::5], axis=0, keepdims=True)
    w2gt = W2g[:, :_GH].T
    w2ge = W2g[:, _GH:].T
    kspec = pl.BlockSpec((_K, _GH), lambda i: (0, 0))
    bspec = pl.BlockSpec((1, _FH), lambda i: (0, 0))
    rkspec = pl.BlockSpec((_ROWS, _K), lambda i: (i, 0))
    out = pl.pallas_call(
        _mlp_block,
        grid=(grid,),
        in_specs=[
            rkspec, rkspec, rkspec,
            pl.BlockSpec((_ROWS, _K * _FD), lambda i: (i, 0)),
            kspec, kspec, kspec, kspec,
            bspec,
            pl.BlockSpec((_GH, _GH), lambda i: (0, 0)),
            pl.BlockSpec((2, _GH), lambda i: (0, 0)),
            pl.BlockSpec((_K * _FD, _FH), lambda i: (0, 0)),
            bspec,
            pl.BlockSpec((_FH, _FH), lambda i: (0, 0)),
            bspec,
        ],
        out_specs=pl.BlockSpec((_ROWS, _GH + _FH), lambda i: (i, 0)),
        out_shape=jax.ShapeDtypeStruct((nrows, _GH + _FH), jnp.float32),
    )(dx, dy, dz, xf, w1x, w1y, w1z, w1e2, b1g, w2gt, w2ge, W1f.T,
      b1f.reshape(1, _FH), W2f.T, b2f.reshape(1, _FH))
    return out


def kernel(xyz, features, W1g, W2g, W1f, b1f, W2f, b2f):
    b, n, _ = xyz.shape
    centroids = _fps_centroids(xyz)
    d2 = (jnp.sum(centroids ** 2, axis=-1)[:, :, None]
          + jnp.sum(xyz ** 2, axis=-1)[:, None, :]
          - 2.0 * jnp.einsum('bmd,bnd->bmn', centroids, xyz))
    idx2 = _sc_topk(d2.reshape(_B * _M * _N)).reshape(_B * _M * _K // 128, 128)
    xf, gx, gy, gz = _sc_gather(features.reshape(_B * _N, _FD), xyz, idx2)

    def _coord(g, c):
        return (g.reshape(b, _M, _K) - centroids[:, :, c:c + 1]).reshape(b * _M, _K)

    dx = _coord(gx, 0)
    dy = _coord(gy, 1)
    dz = _coord(gz, 2)
    out = _mlps(dx, dy, dz, xf.reshape(b * _M, _K * _FD), W1g, W2g, W1f, b1f,
                W2f, b2f)
    return out.reshape(b, _M, _GH + _FH)


# speedup vs baseline: 17.8123x; 1.0627x over previous
import jax
import jax.numpy as jnp
from jax import lax
from jax.experimental import pallas as pl
from jax.experimental.pallas import tpu as pltpu
from jax.experimental.pallas import tpu_sc as plsc

_B, _N, _FD = 8, 4096, 128
_M, _K = 1024, 32
_GH, _FH = 256, 256

_ROWS = 512


def _fps_body(x_ref, y_ref, z_ref, cx_ref, cy_ref, cz_ref):
    X = x_ref[...]
    Y = y_ref[...]
    Z = z_ref[...]
    lane = jax.lax.broadcasted_iota(jnp.int32, (_B, _N), 1)
    big = jnp.int32(_N)

    mlane = jax.lax.broadcasted_iota(jnp.int32, (_B, _M), 1)

    first = lane == 0
    cx0 = jnp.sum(jnp.where(first, X, 0.0), axis=1, keepdims=True)
    cy0 = jnp.sum(jnp.where(first, Y, 0.0), axis=1, keepdims=True)
    cz0 = jnp.sum(jnp.where(first, Z, 0.0), axis=1, keepdims=True)
    zerosm = jnp.zeros((_B, _M), jnp.float32)
    cxs0 = jnp.where(mlane == 0, cx0, zerosm)
    cys0 = jnp.where(mlane == 0, cy0, zerosm)
    czs0 = jnp.where(mlane == 0, cz0, zerosm)

    dists0 = jnp.full((_B, _N), 1e10, dtype=jnp.float32)

    def body(i, state):
        dists, lx, ly, lz, cxs, cys, czs = state
        d = (X - lx) ** 2 + (Y - ly) ** 2 + (Z - lz) ** 2
        dists = jnp.minimum(dists, d)
        m = jnp.max(dists, axis=1, keepdims=True)
        jstar = jnp.min(jnp.where(dists == m, lane, big), axis=1, keepdims=True)
        sel = lane == jstar
        nx = jnp.sum(jnp.where(sel, X, 0.0), axis=1, keepdims=True)
        ny = jnp.sum(jnp.where(sel, Y, 0.0), axis=1, keepdims=True)
        nz = jnp.sum(jnp.where(sel, Z, 0.0), axis=1, keepdims=True)
        here = mlane == i
        cxs = jnp.where(here, nx, cxs)
        cys = jnp.where(here, ny, cys)
        czs = jnp.where(here, nz, czs)
        return dists, nx, ny, nz, cxs, cys, czs

    _, _, _, _, cxs, cys, czs = jax.lax.fori_loop(
        1, _M, body, (dists0, cx0, cy0, cz0, cxs0, cys0, czs0))
    cx_ref[...] = cxs
    cy_ref[...] = cys
    cz_ref[...] = czs


def _fps_centroids(xyz):
    x = xyz[:, :, 0]
    y = xyz[:, :, 1]
    z = xyz[:, :, 2]
    shp = jax.ShapeDtypeStruct((_B, _M), jnp.float32)
    cx, cy, cz = pl.pallas_call(
        _fps_body,
        out_shape=(shp, shp, shp),
    )(x, y, z)
    return jnp.stack([cx, cy, cz], axis=-1)


_NW = 32
_IPW = (_B * _M * _K) // _NW
_FCH = 256
_CCH = 2048


def _gather_body(feat_hbm, xp_hbm, yp_hbm, zp_hbm, idx_hbm,
                 xf_out, gx_out, gy_out, gz_out,
                 idx2_v, gx_v, gy_v, gz_v, rows_v, sem, semf):
    wid = lax.axis_index("s") * 2 + lax.axis_index("c")
    rows_per_w = _IPW // 128
    r0 = wid * rows_per_w

    def super_chunk(c, carry):
        rbase = r0 + c * 16
        pltpu.sync_copy(idx_hbm.at[pl.ds(rbase, 16)], idx2_v)

        def j_body(j, carry2):
            iv = idx2_v.at[j]
            cg1 = pltpu.async_copy(xp_hbm.at[iv], gx_v.at[j], sem)
            cg2 = pltpu.async_copy(yp_hbm.at[iv], gy_v.at[j], sem)
            cg3 = pltpu.async_copy(zp_hbm.at[iv], gz_v.at[j], sem)
            cf = pltpu.async_copy(feat_hbm.at[iv], rows_v, semf)
            cg1.wait()
            cg2.wait()
            cg3.wait()
            cf.wait()
            pltpu.sync_copy(rows_v, xf_out.at[pl.ds((rbase + j) * 128, 128)])
            return carry2

        lax.fori_loop(0, 16, j_body, 0)
        pltpu.sync_copy(gx_v, gx_out.at[pl.ds(rbase, 16)])
        pltpu.sync_copy(gy_v, gy_out.at[pl.ds(rbase, 16)])
        pltpu.sync_copy(gz_v, gz_out.at[pl.ds(rbase, 16)])
        return carry

    lax.fori_loop(0, rows_per_w // 16, super_chunk, 0)


def _sc_gather(features_flat, xyz, idx2):
    mesh = plsc.VectorSubcoreMesh(core_axis_name="c", subcore_axis_name="s")
    nrows = _B * _M * _K
    k = pl.kernel(
        _gather_body,
        mesh=mesh,
        out_type=(
            jax.ShapeDtypeStruct((nrows, _FD), jnp.float32),
            jax.ShapeDtypeStruct((nrows // 128, 128), jnp.float32),
            jax.ShapeDtypeStruct((nrows // 128, 128), jnp.float32),
            jax.ShapeDtypeStruct((nrows // 128, 128), jnp.float32),
        ),
        scratch_types=[
            pltpu.VMEM((16, 128), jnp.int32),
            pltpu.VMEM((16, 128), jnp.float32),
            pltpu.VMEM((16, 128), jnp.float32),
            pltpu.VMEM((16, 128), jnp.float32),
            pltpu.VMEM((128, _FD), jnp.float32),
            pltpu.SemaphoreType.DMA,
            pltpu.SemaphoreType.DMA,
        ],
    )
    xflat = xyz[:, :, 0].reshape(_B * _N)
    yflat = xyz[:, :, 1].reshape(_B * _N)
    zflat = xyz[:, :, 2].reshape(_B * _N)
    return k(features_flat, xflat, yflat, zflat, idx2)


_RPW = (_B * _M) // _NW
_CAP = _N + 16


def _lane_all(v, op):
    i = lax.iota(jnp.int32, 16)
    for sh in (8, 4, 2, 1):
        v = op(v, v.at[i ^ sh].get(mode="promise_in_bounds"))
    return v


def _prefix_sum16(m):
    i = lax.iota(jnp.int32, 16)
    s = m
    for sh in (1, 2, 4, 8):
        g = s.at[jnp.maximum(i - sh, 0)].get(mode="promise_in_bounds")
        s = s + jnp.where(i >= sh, g, 0)
    return s


def _lane_all(v, op):
    i = lax.iota(jnp.int32, 16)
    for sh in (8, 4, 2, 1):
        v = op(v, v.at[i ^ sh].get(mode="promise_in_bounds"))
    return v


_RS = 4


def _topk_body(d2_hbm, out_hbm, row2_v, tmpi_v, out_v, sem):
    wid = lax.axis_index("s") * 2 + lax.axis_index("c")
    r0 = wid * _RPW
    iota16 = lax.iota(jnp.int32, 16)
    inf = jnp.float32(jnp.inf)
    inf_v = jnp.full((16,), inf)
    big_v = jnp.full((16,), jnp.int32(0x7FFFFFFF))
    zi = jnp.zeros((16,), jnp.int32)

    for t in range(_RS):
        pltpu.async_copy(d2_hbm.at[pl.ds((r0 + t) * _N, _N)],
                         row2_v.at[pl.ds(t * _N, _N)], sem)

    def row_body(rg, carry):
        buf = rg % 2
        ra = r0 + _RS * rg
        bases = [pl.multiple_of((buf * _RS + t) * _N, _N) for t in range(_RS)]
        for t in range(_RS):
            pltpu.make_async_copy(
                d2_hbm.at[pl.ds((ra + t) * _N, _N)],
                row2_v.at[pl.ds(bases[t], _N)], sem).wait()

        @pl.when(rg < _RPW // _RS - 1)
        def _prefetch():
            for t in range(_RS):
                nb = pl.multiple_of((((rg + 1) % 2) * _RS + t) * _N, _N)
                pltpu.async_copy(d2_hbm.at[pl.ds((ra + _RS + t) * _N, _N)],
                                 row2_v.at[pl.ds(nb, _N)], sem)

        def sbuild(g, Ss):
            def inner(j, ms):
                return tuple(
                    jnp.minimum(ms[t], row2_v[pl.ds(
                        pl.multiple_of(bases[t] + g * 256 + j * 16, 16), 16)])
                    for t in range(_RS))

            ms = lax.fori_loop(0, 16, inner, (inf_v,) * _RS)
            return tuple(
                jnp.where(iota16 == g, _lane_all(ms[t], jnp.minimum), Ss[t])
                for t in range(_RS))

        S = list(lax.fori_loop(0, 16, sbuild, (inf_v,) * _RS))
        boff = [((ra + t) // _M) * _N for t in range(_RS)]

        o0 = [zi] * _RS
        o1 = [zi] * _RS
        for k in range(_K):
            gm_v = [_lane_all(S[t], jnp.minimum) for t in range(_RS)]
            g_v = [_lane_all(jnp.where(S[t] == gm_v[t], iota16, big_v),
                             jnp.minimum) for t in range(_RS)]
            g = [g_v[t][0] for t in range(_RS)]

            def scan(j, c):
                bp, ng, mx = c
                bp, ng, mx = list(bp), list(ng), list(mx)
                for t in range(_RS):
                    s2 = pl.multiple_of(bases[t] + g[t] * 256 + j * 16, 16)
                    v = row2_v[pl.ds(s2, 16)]
                    eq = v == gm_v[t]
                    bp[t] = jnp.minimum(
                        bp[t],
                        jnp.where(eq, g[t] * 256 + j * 16 + iota16, big_v))
                    ng[t] = ng[t] + jnp.where(eq, jnp.int32(1), jnp.int32(0))
                    mx[t] = jnp.minimum(mx[t], jnp.where(eq, inf_v, v))
                return tuple(bp), tuple(ng), tuple(mx)

            bp, ng, mx = lax.fori_loop(
                0, 16, scan, ((big_v,) * _RS, (zi,) * _RS, (inf_v,) * _RS))
            for t in range(_RS):
                p_v = _lane_all(bp[t], jnp.minimum)
                pcol = p_v[0]
                if k < 16:
                    o0[t] = jnp.where(iota16 == k, p_v + boff[t], o0[t])
                else:
                    o1[t] = jnp.where(iota16 == (k - 16), p_v + boff[t],
                                      o1[t])
                voff = pl.multiple_of(bases[t] + (pcol >> 4) * 16, 16)
                vv = row2_v[pl.ds(voff, 16)]
                row2_v[pl.ds(voff, 16)] = jnp.where(
                    iota16 == (p_v & 15), inf_v, vv)
                ngt = _lane_all(ng[t], jnp.add)
                mexcl = _lane_all(mx[t], jnp.minimum)
                S[t] = jnp.where(iota16 == g_v[t],
                                 jnp.where(ngt > 1, gm_v[t], mexcl), S[t])

        for t in range(_RS):
            out_v[pl.ds(t * _K, 16)] = o0[t]
            out_v[pl.ds(t * _K + 16, 16)] = o1[t]
        pltpu.sync_copy(out_v, out_hbm.at[pl.ds(ra * _K, _RS * _K)])
        return carry

    lax.fori_loop(0, _RPW // _RS, row_body, 0)


def _sc_topk(d2flat):
    mesh = plsc.VectorSubcoreMesh(core_axis_name="c", subcore_axis_name="s")
    k = pl.kernel(
        _topk_body,
        mesh=mesh,
        out_type=jax.ShapeDtypeStruct((_B * _M * _K,), jnp.int32),
        scratch_types=[
            pltpu.VMEM((2 * _RS * _N,), jnp.float32),
            pltpu.VMEM((16,), jnp.int32),
            pltpu.VMEM((_RS * _K,), jnp.int32),
            pltpu.SemaphoreType.DMA,
        ],
    )
    return k(d2flat)


def _mlp_block(dx_ref, dy_ref, dz_ref, xf_ref, w1x_ref, w1y_ref, w1z_ref,
               w1e2_ref, b1g_ref, w2gt_ref, w2ge_ref, w1ft_ref, b1f_ref,
               w2ft_ref, b2f_ref, out_ref):
    dx = dx_ref[...]
    dy = dy_ref[...]
    dz = dz_ref[...]
    e2 = -0.5 * (dx * dx + dy * dy + dz * dz)
    h = (jnp.dot(dx, w1x_ref[...], preferred_element_type=jnp.float32)
         + jnp.dot(dy, w1y_ref[...], preferred_element_type=jnp.float32)
         + jnp.dot(dz, w1z_ref[...], preferred_element_type=jnp.float32)
         + jnp.dot(e2, w1e2_ref[...], preferred_element_type=jnp.float32)
         + b1g_ref[...])
    e2h = -0.5 * jnp.sum(h * h, axis=1, keepdims=True)
    xgeo = (jnp.dot(h, w2gt_ref[...], preferred_element_type=jnp.float32)
            - w2ge_ref[0, :][None, :] + e2h * w2ge_ref[1, :][None, :])
    hf = jnp.maximum(
        jnp.dot(xf_ref[...], w1ft_ref[...], preferred_element_type=jnp.float32)
        + b1f_ref[...], 0.0)
    xfeat = jnp.dot(hf, w2ft_ref[...], preferred_element_type=jnp.float32) + b2f_ref[...]
    out_ref[:, :_GH] = xgeo
    out_ref[:, _GH:] = xfeat


def _mlps(dx, dy, dz, xf, W1g, W2g, W1f, b1f, W2f, b2f):
    nrows = dx.shape[0]
    grid = nrows // _ROWS
    w1gt = W1g.T
    w1x = w1gt[0::5]
    w1y = w1gt[1::5]
    w1z = w1gt[2::5]
    w1e2 = w1gt[4::5]
    b1g = -jnp.sum(w1gt[3::5], axis=0, keepdims=True)
    w2gt = W2g[:, :_GH].T
    w2ge = W2g[:, _GH:].T
    kspec = pl.BlockSpec((_K, _GH), lambda i: (0, 0))
    bspec = pl.BlockSpec((1, _FH), lambda i: (0, 0))
    rkspec = pl.BlockSpec((_ROWS, _K), lambda i: (i, 0))
    out = pl.pallas_call(
        _mlp_block,
        grid=(grid,),
        in_specs=[
            rkspec, rkspec, rkspec,
            pl.BlockSpec((_ROWS, _K * _FD), lambda i: (i, 0)),
            kspec, kspec, kspec, kspec,
            bspec,
            pl.BlockSpec((_GH, _GH), lambda i: (0, 0)),
            pl.BlockSpec((2, _GH), lambda i: (0, 0)),
            pl.BlockSpec((_K * _FD, _FH), lambda i: (0, 0)),
            bspec,
            pl.BlockSpec((_FH, _FH), lambda i: (0, 0)),
            bspec,
        ],
        out_specs=pl.BlockSpec((_ROWS, _GH + _FH), lambda i: (i, 0)),
        out_shape=jax.ShapeDtypeStruct((nrows, _GH + _FH), jnp.float32),
    )(dx, dy, dz, xf, w1x, w1y, w1z, w1e2, b1g, w2gt, w2ge, W1f.T,
      b1f.reshape(1, _FH), W2f.T, b2f.reshape(1, _FH))
    return out


def kernel(xyz, features, W1g, W2g, W1f, b1f, W2f, b2f):
    b, n, _ = xyz.shape
    centroids = _fps_centroids(xyz)
    d2 = (jnp.sum(centroids ** 2, axis=-1)[:, :, None]
          + jnp.sum(xyz ** 2, axis=-1)[:, None, :]
          - 2.0 * jnp.einsum('bmd,bnd->bmn', centroids, xyz))
    idx2 = _sc_topk(d2.reshape(_B * _M * _N)).reshape(_B * _M * _K // 128, 128)
    xf, gx, gy, gz = _sc_gather(features.reshape(_B * _N, _FD), xyz, idx2)

    def _coord(g, c):
        return (g.reshape(b, _M, _K) - centroids[:, :, c:c + 1]).reshape(b * _M, _K)

    dx = _coord(gx, 0)
    dy = _coord(gy, 1)
    dz = _coord(gz, 2)
    out = _mlps(dx, dy, dz, xf.reshape(b * _M, _K * _FD), W1g, W2g, W1f, b1f,
                W2f, b2f)
    return out.reshape(b, _M, _GH + _FH)
